# baseline (speedup 1.0000x reference)
import functools

import jax
import jax.numpy as jnp
from jax import lax
from jax.experimental import pallas as pl
from jax.experimental.pallas import tpu as pltpu
from jax.experimental.pallas import tpu_sc as plsc

Q_EXP = 0.3
K_TRUNC = 0.5
BATCH_N = 4096
CLASSES_N = 1000
TRAIN_N = 50000

NUM_CORES = 2
NUM_SUBCORES = 16
NUM_TILES = NUM_CORES * NUM_SUBCORES
PER_TILE = BATCH_N // NUM_TILES
CHUNKS = PER_TILE // 16
CONST_TERM = (1.0 - K_TRUNC ** Q_EXP) / Q_EXP
LN2 = 0.6931471805599453


def _pow_q(x):
    bits = lax.bitcast_convert_type(x, jnp.int32)
    e = ((bits >> 23) & 0xFF) - 127
    m = lax.bitcast_convert_type((bits & 0x7FFFFF) | 0x3F800000, jnp.float32)
    s = (m - 1.0) / (m + 1.0)
    s2 = s * s
    lnm = 2.0 * s * (1.0 + s2 * (1.0 / 3.0 + s2 * (1.0 / 5.0 + s2 * (1.0 / 7.0 + s2 * (1.0 / 9.0)))))
    lnx = e.astype(jnp.float32) * LN2 + lnm
    return jnp.exp(Q_EXP * lnx)


def _gather_body(logits_f, targets_h, index_h, weight_f, parts_h,
                 tgt_v, idx_v, flat_v, yg_v, w_v, out_v, sem1, sem2):
    wid = lax.axis_index("s") * NUM_CORES + lax.axis_index("c")
    base = wid * PER_TILE
    pltpu.sync_copy(targets_h.at[pl.ds(base, PER_TILE)], tgt_v)
    pltpu.sync_copy(index_h.at[pl.ds(base, PER_TILE)], idx_v)
    for j in range(CHUNKS):
        t16 = tgt_v[pl.ds(j * 16, 16)]
        rows = base + (j * 16) + lax.iota(jnp.int32, 16)
        flat_v[pl.ds(j * 16, 16)] = rows * CLASSES_N + t16
    cp1 = pltpu.async_copy(logits_f.at[flat_v], yg_v, sem1)
    cp2 = pltpu.async_copy(weight_f.at[idx_v], w_v, sem2)
    cp1.wait()
    cp2.wait()
    acc = jnp.zeros((16,), jnp.float32)
    for j in range(CHUNKS):
        yg = yg_v[pl.ds(j * 16, 16)]
        w = w_v[pl.ds(j * 16, 16)]
        acc = acc + ((1.0 - _pow_q(yg)) * (1.0 / Q_EXP) - CONST_TERM) * w
    out_v[...] = acc
    pltpu.sync_copy(out_v, parts_h.at[wid])


_sc_gather = functools.partial(
    pl.kernel,
    out_type=jax.ShapeDtypeStruct((NUM_TILES, 16), jnp.float32),
    mesh=plsc.VectorSubcoreMesh(
        core_axis_name="c", subcore_axis_name="s",
        num_cores=NUM_CORES, num_subcores=NUM_SUBCORES,
    ),
    scratch_types=[
        pltpu.VMEM((PER_TILE,), jnp.int32),
        pltpu.VMEM((PER_TILE,), jnp.int32),
        pltpu.VMEM((PER_TILE,), jnp.int32),
        pltpu.VMEM((PER_TILE,), jnp.float32),
        pltpu.VMEM((PER_TILE,), jnp.float32),
        pltpu.VMEM((16,), jnp.float32),
        pltpu.SemaphoreType.DMA,
        pltpu.SemaphoreType.DMA,
    ],
)(_gather_body)


def _reduce_body(p_ref, o_ref):
    o_ref[...] = jnp.sum(p_ref[...]).reshape(1, 1) * (1.0 / BATCH_N)


_tc_reduce = pl.pallas_call(
    _reduce_body,
    out_shape=jax.ShapeDtypeStruct((1, 1), jnp.float32),
)


def kernel(logits, targets, index, weight):
    parts = _sc_gather(logits.reshape(-1), targets, index, weight.reshape(-1))
    return _tc_reduce(parts)[0, 0]


# speedup vs baseline: 1.2062x; 1.2062x over previous
import functools

import jax
import jax.numpy as jnp
from jax import lax
from jax.experimental import pallas as pl
from jax.experimental.pallas import tpu as pltpu
from jax.experimental.pallas import tpu_sc as plsc

Q_EXP = 0.3
K_TRUNC = 0.5
BATCH_N = 4096
CLASSES_N = 1000
TRAIN_N = 50000

NUM_CORES = 2
NUM_SUBCORES = 16
NUM_TILES = NUM_CORES * NUM_SUBCORES
PER_TILE = BATCH_N // NUM_TILES
CONST_TERM = (1.0 - K_TRUNC ** Q_EXP) / Q_EXP

ROWS_BLK = 256
GRID_N = BATCH_N // ROWS_BLK


def _wgather_body(index_h, weight_f, out_h, idx_v, w_v, sem):
    wid = lax.axis_index("s") * NUM_CORES + lax.axis_index("c")
    base = wid * PER_TILE
    pltpu.sync_copy(index_h.at[pl.ds(base, PER_TILE)], idx_v)
    pltpu.async_copy(weight_f.at[idx_v], w_v, sem).wait()
    pltpu.sync_copy(w_v, out_h.at[pl.ds(base, PER_TILE)])


_sc_wgather = functools.partial(
    pl.kernel,
    out_type=jax.ShapeDtypeStruct((BATCH_N,), jnp.float32),
    mesh=plsc.VectorSubcoreMesh(
        core_axis_name="c", subcore_axis_name="s",
        num_cores=NUM_CORES, num_subcores=NUM_SUBCORES,
    ),
    scratch_types=[
        pltpu.VMEM((PER_TILE,), jnp.int32),
        pltpu.VMEM((PER_TILE,), jnp.float32),
        pltpu.SemaphoreType.DMA,
    ],
)(_wgather_body)


def _loss_body(logits_ref, tgt_ref, w_ref, out_ref):
    t = tgt_ref[0, 0, :]
    wv = w_ref[0, 0, :]
    cols = lax.broadcasted_iota(jnp.int32, (ROWS_BLK, CLASSES_N), 1)
    onehot = cols == t[:, None]
    yg = jnp.sum(jnp.where(onehot, logits_ref[...], 0.0), axis=1)
    g = (1.0 - yg ** Q_EXP) * (1.0 / Q_EXP) - CONST_TERM
    part = jnp.sum(g * wv) * (1.0 / BATCH_N)

    @pl.when(pl.program_id(0) == 0)
    def _():
        out_ref[...] = jnp.zeros_like(out_ref)

    out_ref[...] += part.reshape(1, 1)


_tc_loss = pl.pallas_call(
    _loss_body,
    grid=(GRID_N,),
    in_specs=[
        pl.BlockSpec((ROWS_BLK, CLASSES_N), lambda i: (i, 0)),
        pl.BlockSpec((1, 1, ROWS_BLK), lambda i: (i, 0, 0)),
        pl.BlockSpec((1, 1, ROWS_BLK), lambda i: (i, 0, 0)),
    ],
    out_specs=pl.BlockSpec((1, 1), lambda i: (0, 0)),
    out_shape=jax.ShapeDtypeStruct((1, 1), jnp.float32),
)


def kernel(logits, targets, index, weight):
    w = _sc_wgather(index, weight.reshape(-1))
    out = _tc_loss(
        logits,
        targets.reshape(GRID_N, 1, ROWS_BLK),
        w.reshape(GRID_N, 1, ROWS_BLK),
    )
    return out[0, 0]
